# baseline (speedup 1.0000x reference)
import functools

import jax
import jax.numpy as jnp
from jax import lax
from jax.experimental import pallas as pl
from jax.experimental.pallas import tpu as pltpu
from jax.experimental.pallas import tpu_sc as plsc

_B, _N, _K, _D = 256, 64, 1024, 256
_NT = 8
_STEPS = _N // _NT

_NC, _NS = 2, 16
_NW = _NC * _NS
_BPW = _B // _NW
_ROWS_PW = _BPW * _N
_CHUNK = 128
_NCHUNK = _ROWS_PW // _CHUNK


def _vq_body(wq_ref, cb_ref, idx_ref, oh_ref):
    i = pl.program_id(0)
    niota = lax.broadcasted_iota(jnp.int32, (_B, _N), 1)
    kiota = lax.broadcasted_iota(jnp.int32, (_B, _K), 1)

    acc = jnp.zeros((_B, _N), jnp.int32)
    for j in range(_NT):
        wq = wq_ref[:, j, :]
        cb = cb_ref[j]

        w2 = jnp.sum(wq * wq, axis=1, keepdims=True)
        c2 = jnp.sum(cb * cb, axis=1).reshape(1, _K)
        cross = lax.dot_general(wq, cb, (((1,), (1,)), ((), ())),
                                preferred_element_type=jnp.float32)
        dist = w2 - 2.0 * cross + c2

        m = jnp.min(dist, axis=1, keepdims=True)
        idx_col = jnp.min(jnp.where(dist == m, kiota, _K), axis=1,
                          keepdims=True)
        oh_ref[:, j, :] = (kiota == idx_col).astype(jnp.float32)
        acc += jnp.where(niota == i * _NT + j, idx_col, 0)

    @pl.when(i == 0)
    def _():
        idx_ref[...] = acc

    @pl.when(i != 0)
    def _():
        idx_ref[...] += acc


def _tc_quantize(w_q, codebook):
    return pl.pallas_call(
        _vq_body,
        grid=(_STEPS,),
        in_specs=[
            pl.BlockSpec((_B, _NT, _D), lambda i: (0, i, 0)),
            pl.BlockSpec((_NT, _K, _D), lambda i: (i, 0, 0)),
        ],
        out_specs=[
            pl.BlockSpec((_B, _N), lambda i: (0, 0)),
            pl.BlockSpec((_B, _NT, _K), lambda i: (0, i, 0)),
        ],
        out_shape=[
            jax.ShapeDtypeStruct((_B, _N), jnp.int32),
            jax.ShapeDtypeStruct((_B, _N, _K), jnp.float32),
        ],
    )(w_q, codebook)


@functools.partial(
    pl.kernel,
    mesh=plsc.VectorSubcoreMesh(core_axis_name="c", subcore_axis_name="s"),
    out_type=jax.ShapeDtypeStruct((_B * _N, _D), jnp.float32),
    scratch_types=[
        pltpu.VMEM((_BPW, _N), jnp.int32),
        pltpu.VMEM((_NCHUNK, _CHUNK), jnp.int32),
        pltpu.VMEM((_CHUNK, _D), jnp.float32),
        pltpu.SemaphoreType.DMA,
    ],
)
def _sc_gather(table_hbm, idx_hbm, out_hbm, idx2d, flat_idx, rows, sem):
    wid = lax.axis_index("s") * _NC + lax.axis_index("c")
    base_b = wid * _BPW
    pltpu.sync_copy(idx_hbm.at[pl.ds(base_b, _BPW), :], idx2d)
    lane = lax.iota(jnp.int32, 16)
    for r in range(_BPW):
        for j in range(_N // 16):
            n_off = (lane + j * 16) * _K
            v = idx2d[r, pl.ds(j * 16, 16)] + n_off
            flat = r * _N + j * 16
            flat_idx[flat // _CHUNK, pl.ds(flat % _CHUNK, 16)] = v
    for c in range(_NCHUNK):
        pltpu.async_copy(table_hbm.at[flat_idx.at[c]], rows, sem).wait()
        pltpu.sync_copy(
            rows, out_hbm.at[pl.ds(wid * _ROWS_PW + c * _CHUNK, _CHUNK), :])


def kernel(w_q, codebook):
    idx, one_hot = _tc_quantize(w_q, codebook)
    table = codebook.reshape(_N * _K, _D)
    w2d = _sc_gather(table, idx)
    return (w2d.reshape(_B, _N, _D), idx, one_hot)
